# baseline (speedup 1.0000x reference)
import functools

import jax
import jax.numpy as jnp
from jax import lax
from jax.experimental import pallas as pl
from jax.experimental.pallas import tpu as pltpu
from jax.experimental.pallas import tpu_sc as plsc

_B, _D, _L, _K = 9216, 768, 256, 8192
_BLK = 256

_NW = 32
_KH = _K // 2


def _encode_vq_body(x_ref, w1_ref, b1_ref, w2_ref, b2_ref, w3_ref, b3_ref,
                    cb_ref, ze_ref, idx_ref, csum_ref):
    i = pl.program_id(0)

    @pl.when(i == 0)
    def _():
        cb = cb_ref[...]
        csum_ref[...] = jnp.sum(cb * cb, axis=1)[None, :]

    x = x_ref[...]
    h = jnp.maximum(jnp.dot(x, w1_ref[...]) + b1_ref[...], 0.0)
    h = jnp.maximum(jnp.dot(h, w2_ref[...]) + b2_ref[...], 0.0)
    z = jnp.dot(h, w3_ref[...]) + b3_ref[...]
    ze_ref[...] = z

    ab2 = lax.dot_general(-2.0 * z, cb_ref[...], (((1,), (1,)), ((), ())))
    rowsum = jnp.sum(z * z, axis=1, keepdims=True)
    d2 = (rowsum + ab2) + csum_ref[...]
    m = jnp.min(d2, axis=1, keepdims=True)
    col = lax.broadcasted_iota(jnp.int32, (_BLK, _K), 1)
    idx = jnp.min(jnp.where(d2 == m, col, _K), axis=1)
    idx_ref[...] = idx[None, None, :]


def _encode_vq(x, w1, b1, w2, b2, w3, b3, cb):
    n = x.shape[0]
    nblk = n // _BLK
    return pl.pallas_call(
        _encode_vq_body,
        grid=(nblk,),
        in_specs=[
            pl.BlockSpec((_BLK, _D), lambda i: (i, 0)),
            pl.BlockSpec((_D, 256), lambda i: (0, 0)),
            pl.BlockSpec((256,), lambda i: (0,)),
            pl.BlockSpec((256, 256), lambda i: (0, 0)),
            pl.BlockSpec((256,), lambda i: (0,)),
            pl.BlockSpec((256, _L), lambda i: (0, 0)),
            pl.BlockSpec((_L,), lambda i: (0,)),
            pl.BlockSpec((_K, _L), lambda i: (0, 0)),
        ],
        out_specs=[
            pl.BlockSpec((_BLK, _L), lambda i: (i, 0)),
            pl.BlockSpec((1, 1, _BLK), lambda i: (i, 0, 0)),
        ],
        out_shape=[
            jax.ShapeDtypeStruct((n, _L), jnp.float32),
            jax.ShapeDtypeStruct((nblk, 1, _BLK), jnp.int32),
        ],
        scratch_shapes=[pltpu.VMEM((1, _K), jnp.float32)],
    )(x, w1, b1, w2, b2, w3, b3, cb)


def _sc_gather(codebook, idx_flat):
    n = idx_flat.shape[0]
    bpw = n // _NW
    sb = 96 if bpw % 96 == 0 else 72
    nsb = bpw // sb
    mesh = plsc.VectorSubcoreMesh(core_axis_name="c", subcore_axis_name="s")

    @functools.partial(
        pl.kernel,
        out_type=jax.ShapeDtypeStruct((n, _L), jnp.float32),
        mesh=mesh,
        compiler_params=pltpu.CompilerParams(use_tc_tiling_on_sc=False,
                                             needs_layout_passes=False),
        scratch_types=[
            pltpu.VMEM_SHARED((_KH, _L), jnp.float32),
            pltpu.VMEM_SHARED((n,), jnp.int32),
            pltpu.VMEM((sb, _L), jnp.float32),
            pltpu.VMEM((sb, _L), jnp.float32),
            pltpu.VMEM((sb,), jnp.int32),
            pltpu.SMEM((bpw,), jnp.int32),
            pltpu.SemaphoreType.DMA,
        ],
    )
    def k(cb_hbm, idx_hbm, out_hbm, table_sp, idx_sp, buf_a, buf_b, idx_v,
          idx_sm, sem):
        sid = lax.axis_index("s")
        wid = sid * 2 + lax.axis_index("c")
        base = wid * bpw

        @pl.when(sid == 0)
        def _():
            pltpu.sync_copy(idx_hbm, idx_sp)

        plsc.subcore_barrier()
        pltpu.sync_copy(idx_sp.at[pl.ds(base, bpw)], idx_sm)

        part = _KH // 16

        def stage(half_base):
            pltpu.sync_copy(
                cb_hbm.at[pl.ds(half_base + sid * part, part)],
                table_sp.at[pl.ds(sid * part, part)],
            )

        for s in range(nsb):
            stage(0)
            plsc.subcore_barrier()
            pltpu.sync_copy(idx_hbm.at[pl.ds(base + s * sb, sb)], idx_v)
            for t in range(sb // 16):
                sl = pl.ds(t * 16, 16)
                idx_v[sl] = jnp.minimum(idx_v[sl], _KH - 1)
            pltpu.async_copy(table_sp.at[idx_v], buf_a, sem).wait()
            plsc.subcore_barrier()

            stage(_KH)
            plsc.subcore_barrier()
            pltpu.sync_copy(idx_hbm.at[pl.ds(base + s * sb, sb)], idx_v)
            for t in range(sb // 16):
                sl = pl.ds(t * 16, 16)
                idx_v[sl] = jnp.maximum(idx_v[sl] - _KH, 0)
            pltpu.async_copy(table_sp.at[idx_v], buf_b, sem).wait()

            def body(i, carry):
                @pl.when(idx_sm[s * sb + i] >= _KH)
                def _():
                    for c in range(_L // 16):
                        cs = pl.ds(c * 16, 16)
                        buf_a[i, cs] = buf_b[i, cs]

                return carry

            lax.fori_loop(0, sb, body, 0)

            pltpu.sync_copy(buf_a, out_hbm.at[pl.ds(base + s * sb, sb)])
            plsc.subcore_barrier()

    return k(codebook, idx_flat)


def _decode_body(ze_ref, zq_ref, w1_ref, b1_ref, w2_ref, b2_ref, w3_ref,
                 b3_ref, xrec_ref, loss_ref, acc_ref):
    i = pl.program_id(0)
    ze = ze_ref[...]
    zq = zq_ref[...]
    zst = ze + (zq - ze)
    h = jnp.maximum(jnp.dot(zst, w1_ref[...]) + b1_ref[...], 0.0)
    h = jnp.maximum(jnp.dot(h, w2_ref[...]) + b2_ref[...], 0.0)
    xrec_ref[...] = jnp.dot(h, w3_ref[...]) + b3_ref[...]

    diff = ze - zq
    part = jnp.sum(diff * diff)

    @pl.when(i == 0)
    def _():
        acc_ref[0] = 0.0

    acc_ref[0] += part

    @pl.when(i == pl.num_programs(0) - 1)
    def _():
        loss_ref[...] = acc_ref[0].reshape(1, 1)


def _decode(ze, zq, w1, b1, w2, b2, w3, b3):
    n = ze.shape[0]
    nblk = n // _BLK
    return pl.pallas_call(
        _decode_body,
        grid=(nblk,),
        in_specs=[
            pl.BlockSpec((_BLK, _L), lambda i: (i, 0)),
            pl.BlockSpec((_BLK, _L), lambda i: (i, 0)),
            pl.BlockSpec((_L, 256), lambda i: (0, 0)),
            pl.BlockSpec((256,), lambda i: (0,)),
            pl.BlockSpec((256, 256), lambda i: (0, 0)),
            pl.BlockSpec((256,), lambda i: (0,)),
            pl.BlockSpec((256, _D), lambda i: (0, 0)),
            pl.BlockSpec((_D,), lambda i: (0,)),
        ],
        out_specs=[
            pl.BlockSpec((_BLK, _D), lambda i: (i, 0)),
            pl.BlockSpec((1, 1), lambda i: (0, 0)),
        ],
        out_shape=[
            jax.ShapeDtypeStruct((n, _D), jnp.float32),
            jax.ShapeDtypeStruct((1, 1), jnp.float32),
        ],
        scratch_shapes=[pltpu.SMEM((1,), jnp.float32)],
    )(ze, zq, w1, b1, w2, b2, w3, b3)


_NSLICE = 2


def kernel(x, enc_w1, enc_b1, enc_w2, enc_b2, enc_w3, enc_b3,
           dec_w1, dec_b1, dec_w2, dec_b2, dec_w3, dec_b3, codebook):
    ns = _B // _NSLICE
    zes, idxs, zqs = [], [], []
    for si in range(_NSLICE):
        xs = lax.slice_in_dim(x, si * ns, (si + 1) * ns, axis=0)
        ze, idx3 = _encode_vq(xs, enc_w1, enc_b1, enc_w2, enc_b2, enc_w3,
                              enc_b3, codebook)
        idx_flat = idx3.reshape(ns)
        zq = _sc_gather(codebook, idx_flat)
        zes.append(ze)
        idxs.append(idx_flat)
        zqs.append(zq)

    xrecs, parts = [], []
    for si in range(_NSLICE):
        xrec, part = _decode(zes[si], zqs[si], dec_w1, dec_b1, dec_w2,
                             dec_b2, dec_w3, dec_b3)
        xrecs.append(xrec)
        parts.append(part)

    xrec = jnp.concatenate(xrecs, axis=0)
    zq = jnp.concatenate(zqs, axis=0)
    idx = jnp.concatenate(idxs, axis=0)
    loss = sum(p.reshape(()) for p in parts) / jnp.float32(_B * _L)
    return xrec, zq, loss, idx


# speedup vs baseline: 1.1395x; 1.1395x over previous
import functools

import jax
import jax.numpy as jnp
from jax import lax
from jax.experimental import pallas as pl
from jax.experimental.pallas import tpu as pltpu
from jax.experimental.pallas import tpu_sc as plsc

_B, _D, _L, _K = 9216, 768, 256, 8192
_BLK = 256

_NW = 32
_KH = _K // 2


def _encode_vq_body(x_ref, w1_ref, b1_ref, w2_ref, b2_ref, w3_ref, b3_ref,
                    cb_ref, ze_ref, idx_ref, csum_ref):
    i = pl.program_id(0)

    @pl.when(i == 0)
    def _():
        cb = cb_ref[...]
        csum_ref[...] = jnp.sum(cb * cb, axis=1)[None, :]

    x = x_ref[...]
    h = jnp.maximum(jnp.dot(x, w1_ref[...]) + b1_ref[...], 0.0)
    h = jnp.maximum(jnp.dot(h, w2_ref[...]) + b2_ref[...], 0.0)
    z = jnp.dot(h, w3_ref[...]) + b3_ref[...]
    ze_ref[...] = z

    ab2 = lax.dot_general(-2.0 * z, cb_ref[...], (((1,), (1,)), ((), ())))
    rowsum = jnp.sum(z * z, axis=1, keepdims=True)
    d2 = (rowsum + ab2) + csum_ref[...]
    m = jnp.min(d2, axis=1, keepdims=True)
    col = lax.broadcasted_iota(jnp.int32, (_BLK, _K), 1)
    idx = jnp.min(jnp.where(d2 == m, col, _K), axis=1)
    idx_ref[...] = idx[None, None, :]


def _encode_vq(x, w1, b1, w2, b2, w3, b3, cb):
    n = x.shape[0]
    nblk = n // _BLK
    return pl.pallas_call(
        _encode_vq_body,
        grid=(nblk,),
        in_specs=[
            pl.BlockSpec((_BLK, _D), lambda i: (i, 0)),
            pl.BlockSpec((_D, 256), lambda i: (0, 0)),
            pl.BlockSpec((256,), lambda i: (0,)),
            pl.BlockSpec((256, 256), lambda i: (0, 0)),
            pl.BlockSpec((256,), lambda i: (0,)),
            pl.BlockSpec((256, _L), lambda i: (0, 0)),
            pl.BlockSpec((_L,), lambda i: (0,)),
            pl.BlockSpec((_K, _L), lambda i: (0, 0)),
        ],
        out_specs=[
            pl.BlockSpec((_BLK, _L), lambda i: (i, 0)),
            pl.BlockSpec((1, 1, _BLK), lambda i: (i, 0, 0)),
        ],
        out_shape=[
            jax.ShapeDtypeStruct((n, _L), jnp.float32),
            jax.ShapeDtypeStruct((nblk, 1, _BLK), jnp.int32),
        ],
        scratch_shapes=[pltpu.VMEM((1, _K), jnp.float32)],
    )(x, w1, b1, w2, b2, w3, b3, cb)


def _sc_gather(codebook, idx_flat):
    n = idx_flat.shape[0]
    bpw = n // _NW
    sb = 96 if bpw % 96 == 0 else 72
    nsb = bpw // sb
    mesh = plsc.VectorSubcoreMesh(core_axis_name="c", subcore_axis_name="s")

    @functools.partial(
        pl.kernel,
        out_type=jax.ShapeDtypeStruct((n, _L), jnp.float32),
        mesh=mesh,
        compiler_params=pltpu.CompilerParams(use_tc_tiling_on_sc=False,
                                             needs_layout_passes=False),
        scratch_types=[
            pltpu.VMEM_SHARED((_KH, _L), jnp.float32),
            pltpu.VMEM_SHARED((n,), jnp.int32),
            pltpu.VMEM((sb, _L), jnp.float32),
            pltpu.VMEM((sb, _L), jnp.float32),
            pltpu.VMEM((sb,), jnp.int32),
            pltpu.SMEM((bpw,), jnp.int32),
            pltpu.SemaphoreType.DMA,
        ],
    )
    def k(cb_hbm, idx_hbm, out_hbm, table_sp, idx_sp, buf_a, buf_b, idx_v,
          idx_sm, sem):
        sid = lax.axis_index("s")
        wid = sid * 2 + lax.axis_index("c")
        base = wid * bpw

        @pl.when(sid == 0)
        def _():
            pltpu.sync_copy(idx_hbm, idx_sp)

        plsc.subcore_barrier()
        pltpu.sync_copy(idx_sp.at[pl.ds(base, bpw)], idx_sm)

        part = _KH // 16

        def stage(half_base):
            pltpu.sync_copy(
                cb_hbm.at[pl.ds(half_base + sid * part, part)],
                table_sp.at[pl.ds(sid * part, part)],
            )

        for s in range(nsb):
            stage(0)
            plsc.subcore_barrier()
            pltpu.sync_copy(idx_hbm.at[pl.ds(base + s * sb, sb)], idx_v)
            for t in range(sb // 16):
                sl = pl.ds(t * 16, 16)
                idx_v[sl] = jnp.minimum(idx_v[sl], _KH - 1)
            pltpu.async_copy(table_sp.at[idx_v], buf_a, sem).wait()
            plsc.subcore_barrier()

            stage(_KH)
            plsc.subcore_barrier()
            pltpu.sync_copy(idx_hbm.at[pl.ds(base + s * sb, sb)], idx_v)
            for t in range(sb // 16):
                sl = pl.ds(t * 16, 16)
                idx_v[sl] = jnp.maximum(idx_v[sl] - _KH, 0)
            pltpu.async_copy(table_sp.at[idx_v], buf_b, sem).wait()

            def body(i, carry):
                @pl.when(idx_sm[s * sb + i] >= _KH)
                def _():
                    for c in range(_L // 16):
                        cs = pl.ds(c * 16, 16)
                        buf_a[i, cs] = buf_b[i, cs]

                return carry

            lax.fori_loop(0, sb, body, 0)

            pltpu.sync_copy(buf_a, out_hbm.at[pl.ds(base + s * sb, sb)])
            plsc.subcore_barrier()

    return k(codebook, idx_flat)


def _decode_body(ze_ref, zq_ref, w1_ref, b1_ref, w2_ref, b2_ref, w3_ref,
                 b3_ref, xrec_ref, loss_ref, acc_ref):
    i = pl.program_id(0)
    ze = ze_ref[...]
    zq = zq_ref[...]
    zst = ze + (zq - ze)
    h = jnp.maximum(jnp.dot(zst, w1_ref[...]) + b1_ref[...], 0.0)
    h = jnp.maximum(jnp.dot(h, w2_ref[...]) + b2_ref[...], 0.0)
    xrec_ref[...] = jnp.dot(h, w3_ref[...]) + b3_ref[...]

    diff = ze - zq
    part = jnp.sum(diff * diff)

    @pl.when(i == 0)
    def _():
        acc_ref[0] = 0.0

    acc_ref[0] += part

    @pl.when(i == pl.num_programs(0) - 1)
    def _():
        loss_ref[...] = acc_ref[0].reshape(1, 1)


def _decode(ze, zq, w1, b1, w2, b2, w3, b3):
    n = ze.shape[0]
    nblk = n // _BLK
    return pl.pallas_call(
        _decode_body,
        grid=(nblk,),
        in_specs=[
            pl.BlockSpec((_BLK, _L), lambda i: (i, 0)),
            pl.BlockSpec((_BLK, _L), lambda i: (i, 0)),
            pl.BlockSpec((_L, 256), lambda i: (0, 0)),
            pl.BlockSpec((256,), lambda i: (0,)),
            pl.BlockSpec((256, 256), lambda i: (0, 0)),
            pl.BlockSpec((256,), lambda i: (0,)),
            pl.BlockSpec((256, _D), lambda i: (0, 0)),
            pl.BlockSpec((_D,), lambda i: (0,)),
        ],
        out_specs=[
            pl.BlockSpec((_BLK, _D), lambda i: (i, 0)),
            pl.BlockSpec((1, 1), lambda i: (0, 0)),
        ],
        out_shape=[
            jax.ShapeDtypeStruct((n, _D), jnp.float32),
            jax.ShapeDtypeStruct((1, 1), jnp.float32),
        ],
        scratch_shapes=[pltpu.SMEM((1,), jnp.float32)],
    )(ze, zq, w1, b1, w2, b2, w3, b3)


def kernel(x, enc_w1, enc_b1, enc_w2, enc_b2, enc_w3, enc_b3,
           dec_w1, dec_b1, dec_w2, dec_b2, dec_w3, dec_b3, codebook):
    ze, idx3 = _encode_vq(x, enc_w1, enc_b1, enc_w2, enc_b2, enc_w3,
                          enc_b3, codebook)
    idx_flat = idx3.reshape(_B)
    zq = _sc_gather(codebook, idx_flat)
    xrec, part = _decode(ze, zq, dec_w1, dec_b1, dec_w2, dec_b2,
                         dec_w3, dec_b3)
    loss = part.reshape(()) / jnp.float32(_B * _L)
    return xrec, zq, loss, idx_flat
